# baseline (speedup 1.0000x reference)
import functools

import jax
import jax.numpy as jnp
from jax import lax
from jax.experimental import pallas as pl
from jax.experimental.pallas import tpu as pltpu
from jax.experimental.pallas import tpu_sc as plsc

DIM = 2048
N_EXPERTS = 64
TOP_K = 6
LANES = 16

BT = 2048
N_WORKERS = 32


def _mm_softmax_body(x_ref, w_ref, p_ref):
    sT = jax.lax.dot_general(
        w_ref[...], x_ref[...], (((1,), (1,)), ((), ())),
        preferred_element_type=jnp.float32,
    )
    m = jnp.max(sT, axis=0, keepdims=True)
    e = jnp.exp(sT - m)
    pT = e / jnp.sum(e, axis=0, keepdims=True)
    p_ref[...] = pT.T


def _tc_probs(x, weight):
    n = x.shape[0]
    return pl.pallas_call(
        _mm_softmax_body,
        grid=(n // BT,),
        in_specs=[
            pl.BlockSpec((BT, DIM), lambda i: (i, 0)),
            pl.BlockSpec((N_EXPERTS, DIM), lambda i: (0, 0)),
        ],
        out_specs=pl.BlockSpec((BT, N_EXPERTS), lambda i: (i, 0)),
        out_shape=jax.ShapeDtypeStruct((n, N_EXPERTS), jnp.float32),
    )(x, weight)


def _merge16(k1, v1, k2, v2):
    k2r = lax.rev(k2, (0,))
    v2r = lax.rev(v2, (0,))
    take1 = k1 >= k2r
    km = jnp.where(take1, k1, k2r)
    vm = jnp.where(take1, v1, v2r)
    return plsc.sort_key_val(km, vm, descending=True)


def _sc_topk_body(rows_per, p_hbm, bias_hbm, wout_hbm, iout_hbm,
                  p_v, bias_v, w_v, i_v):
    wid = lax.axis_index("s") * 2 + lax.axis_index("c")
    base = wid * rows_per
    pltpu.sync_copy(p_hbm.at[pl.ds(base, rows_per)], p_v)
    pltpu.sync_copy(bias_hbm, bias_v)
    biases = [bias_v[pl.ds(LANES * j, LANES)] for j in range(4)]
    iotas = [lax.iota(jnp.int32, LANES) + LANES * j for j in range(4)]

    iota = lax.iota(jnp.int32, LANES)
    lo8 = iota < 8
    shf1 = jnp.where(lo8, 7 - iota, 0)
    shf2 = jnp.where(lo8, 0, iota - 8)
    shf3 = 15 - iota

    def _take(x, idx):
        return x.at[idx].get(mode="promise_in_bounds")

    @plsc.parallel_loop(0, rows_per, unroll=8)
    def row_fn(r):
        segs = [p_v[r, pl.ds(LANES * j, LANES)] for j in range(4)]
        kv = [
            plsc.sort_key_val(segs[j] + biases[j], iotas[j], descending=True)
            for j in range(4)
        ]
        (k0, v0), (k1, v1), (k2, v2), (k3, v3) = kv
        g1k, g1v = _take(k1, shf1), _take(v1, shf1)
        g2k, g2v = _take(k2, shf2), _take(v2, shf2)
        g3k, g3v = _take(k3, shf3), _take(v3, shf3)
        t01 = k0 >= g1k
        ak, av = jnp.where(t01, k0, g1k), jnp.where(t01, v0, g1v)
        t23 = g2k >= g3k
        bk, bv = jnp.where(t23, g2k, g3k), jnp.where(t23, g2v, g3v)
        ck, cv = jnp.where(lo8, ak, bk), jnp.where(lo8, av, bv)
        _, vf = plsc.sort_key_val(ck, cv, descending=True)
        rvec = jnp.full((LANES,), r, jnp.int32)
        w_v[r, :] = plsc.load_gather(p_v, [rvec, vf])
        i_v[r, :] = vf
    pltpu.sync_copy(w_v, wout_hbm.at[pl.ds(base, rows_per)])
    pltpu.sync_copy(i_v, iout_hbm.at[pl.ds(base, rows_per)])


def _sc_topk(probs, bias):
    n = probs.shape[0]
    rows_per = n // N_WORKERS
    mesh = plsc.VectorSubcoreMesh(core_axis_name="c", subcore_axis_name="s")
    return pl.kernel(
        functools.partial(_sc_topk_body, rows_per),
        out_type=[
            jax.ShapeDtypeStruct((n, LANES), jnp.float32),
            jax.ShapeDtypeStruct((n, LANES), jnp.int32),
        ],
        mesh=mesh,
        scratch_types=[
            pltpu.VMEM((rows_per, N_EXPERTS), jnp.float32),
            pltpu.VMEM((N_EXPERTS,), jnp.float32),
            pltpu.VMEM((rows_per, LANES), jnp.float32),
            pltpu.VMEM((rows_per, LANES), jnp.int32),
        ],
        compiler_params=pltpu.CompilerParams(needs_layout_passes=False),
    )(probs, bias)


@jax.jit
def kernel(x, weight, bias):
    probs = _tc_probs(x, weight)
    wpad, ipad = _sc_topk(probs, bias)
    return wpad[:, :TOP_K].astype(x.dtype), ipad[:, :TOP_K]


# speedup vs baseline: 1.0324x; 1.0324x over previous
import functools

import jax
import jax.numpy as jnp
from jax import lax
from jax.experimental import pallas as pl
from jax.experimental.pallas import tpu as pltpu
from jax.experimental.pallas import tpu_sc as plsc

DIM = 2048
N_EXPERTS = 64
TOP_K = 6
LANES = 16

BT = 1024
N_WORKERS = 32


def _mm_softmax_body(x_ref, w_ref, p_ref):
    sT = jax.lax.dot_general(
        w_ref[...], x_ref[...], (((1,), (1,)), ((), ())),
        preferred_element_type=jnp.float32,
    )
    m = jnp.max(sT, axis=0, keepdims=True)
    e = jnp.exp(sT - m)
    pT = e / jnp.sum(e, axis=0, keepdims=True)
    p_ref[...] = pT.T


def _tc_probs(x, weight):
    n = x.shape[0]
    return pl.pallas_call(
        _mm_softmax_body,
        grid=(n // BT,),
        in_specs=[
            pl.BlockSpec((BT, DIM), lambda i: (i, 0)),
            pl.BlockSpec((N_EXPERTS, DIM), lambda i: (0, 0)),
        ],
        out_specs=pl.BlockSpec((BT, N_EXPERTS), lambda i: (i, 0)),
        out_shape=jax.ShapeDtypeStruct((n, N_EXPERTS), jnp.float32),
    )(x, weight)


def _merge16(k1, v1, k2, v2):
    k2r = lax.rev(k2, (0,))
    v2r = lax.rev(v2, (0,))
    take1 = k1 >= k2r
    km = jnp.where(take1, k1, k2r)
    vm = jnp.where(take1, v1, v2r)
    return plsc.sort_key_val(km, vm, descending=True)


def _sc_topk_body(rows_per, p_hbm, bias_hbm, wout_hbm, iout_hbm,
                  p_v, bias_v, w_v, i_v):
    wid = lax.axis_index("s") * 2 + lax.axis_index("c")
    base = wid * rows_per
    pltpu.sync_copy(p_hbm.at[pl.ds(base, rows_per)], p_v)
    pltpu.sync_copy(bias_hbm, bias_v)
    biases = [bias_v[pl.ds(LANES * j, LANES)] for j in range(4)]
    iotas = [lax.iota(jnp.int32, LANES) + LANES * j for j in range(4)]

    iota = lax.iota(jnp.int32, LANES)
    lo8 = iota < 8
    shf1 = jnp.where(lo8, 7 - iota, 0)
    shf2 = jnp.where(lo8, 0, iota - 8)
    shf3 = 15 - iota

    def _take(x, idx):
        return x.at[idx].get(mode="promise_in_bounds")

    @plsc.parallel_loop(0, rows_per, unroll=4)
    def row_fn(r):
        segs = [p_v[r, pl.ds(LANES * j, LANES)] for j in range(4)]
        kv = [
            plsc.sort_key_val(segs[j] + biases[j], iotas[j], descending=True)
            for j in range(4)
        ]
        (k0, v0), (k1, v1), (k2, v2), (k3, v3) = kv
        g1k, g1v = _take(k1, shf1), _take(v1, shf1)
        g2k, g2v = _take(k2, shf2), _take(v2, shf2)
        g3k, g3v = _take(k3, shf3), _take(v3, shf3)
        t01 = k0 >= g1k
        ak, av = jnp.where(t01, k0, g1k), jnp.where(t01, v0, g1v)
        t23 = g2k >= g3k
        bk, bv = jnp.where(t23, g2k, g3k), jnp.where(t23, g2v, g3v)
        ck, cv = jnp.where(lo8, ak, bk), jnp.where(lo8, av, bv)
        _, vf = plsc.sort_key_val(ck, cv, descending=True)
        rvec = jnp.full((LANES,), r, jnp.int32)
        w_v[r, :] = plsc.load_gather(p_v, [rvec, vf])
        i_v[r, :] = vf
    pltpu.sync_copy(w_v, wout_hbm.at[pl.ds(base, rows_per)])
    pltpu.sync_copy(i_v, iout_hbm.at[pl.ds(base, rows_per)])


def _sc_topk(probs, bias):
    n = probs.shape[0]
    rows_per = n // N_WORKERS
    mesh = plsc.VectorSubcoreMesh(core_axis_name="c", subcore_axis_name="s")
    return pl.kernel(
        functools.partial(_sc_topk_body, rows_per),
        out_type=[
            jax.ShapeDtypeStruct((n, LANES), jnp.float32),
            jax.ShapeDtypeStruct((n, LANES), jnp.int32),
        ],
        mesh=mesh,
        scratch_types=[
            pltpu.VMEM((rows_per, N_EXPERTS), jnp.float32),
            pltpu.VMEM((N_EXPERTS,), jnp.float32),
            pltpu.VMEM((rows_per, LANES), jnp.float32),
            pltpu.VMEM((rows_per, LANES), jnp.int32),
        ],
        compiler_params=pltpu.CompilerParams(needs_layout_passes=False),
    )(probs, bias)


@jax.jit
def kernel(x, weight, bias):
    probs = _tc_probs(x, weight)
    wpad, ipad = _sc_topk(probs, bias)
    return wpad[:, :TOP_K].astype(x.dtype), ipad[:, :TOP_K]


# speedup vs baseline: 1.0364x; 1.0039x over previous
import functools

import jax
import jax.numpy as jnp
from jax import lax
from jax.experimental import pallas as pl
from jax.experimental.pallas import tpu as pltpu
from jax.experimental.pallas import tpu_sc as plsc

DIM = 2048
N_EXPERTS = 64
TOP_K = 6
LANES = 16

BT = 1024
N_WORKERS = 32


def _mm_softmax_body(x_ref, w_ref, p_ref):
    sT = jax.lax.dot_general(
        w_ref[...], x_ref[...], (((1,), (1,)), ((), ())),
        preferred_element_type=jnp.float32,
    )
    m = jnp.max(sT, axis=0, keepdims=True)
    e = jnp.exp(sT - m)
    pT = e / jnp.sum(e, axis=0, keepdims=True)
    p_ref[...] = pT.T


def _tc_probs(x, weight):
    n = x.shape[0]
    return pl.pallas_call(
        _mm_softmax_body,
        grid=(n // BT,),
        in_specs=[
            pl.BlockSpec((BT, DIM), lambda i: (i, 0)),
            pl.BlockSpec((N_EXPERTS, DIM), lambda i: (0, 0)),
        ],
        out_specs=pl.BlockSpec((BT, N_EXPERTS), lambda i: (i, 0)),
        out_shape=jax.ShapeDtypeStruct((n, N_EXPERTS), jnp.float32),
    )(x, weight)


def _merge16(k1, v1, k2, v2):
    k2r = lax.rev(k2, (0,))
    v2r = lax.rev(v2, (0,))
    take1 = k1 >= k2r
    km = jnp.where(take1, k1, k2r)
    vm = jnp.where(take1, v1, v2r)
    return plsc.sort_key_val(km, vm, descending=True)


def _sc_topk_body(rows_per, p_hbm, bias_hbm, wout_hbm, iout_hbm,
                  p_v, bias_v, w_v, i_v):
    wid = lax.axis_index("s") * 2 + lax.axis_index("c")
    base = wid * rows_per
    pltpu.sync_copy(p_hbm.at[pl.ds(base, rows_per)], p_v)
    pltpu.sync_copy(bias_hbm, bias_v)
    biases = [bias_v[pl.ds(LANES * j, LANES)] for j in range(4)]
    iotas = [lax.iota(jnp.int32, LANES) + LANES * j for j in range(4)]

    iota = lax.iota(jnp.int32, LANES)
    lo8 = iota < 8
    shf1 = jnp.where(lo8, 7 - iota, 0)
    shf2 = jnp.where(lo8, 0, iota - 8)
    shf3 = 15 - iota

    def _take(x, idx):
        return x.at[idx].get(mode="promise_in_bounds")

    @plsc.parallel_loop(0, rows_per, unroll=2)
    def row_fn(r):
        segs = [p_v[r, pl.ds(LANES * j, LANES)] for j in range(4)]
        kv = [
            plsc.sort_key_val(segs[j] + biases[j], iotas[j], descending=True)
            for j in range(4)
        ]
        (k0, v0), (k1, v1), (k2, v2), (k3, v3) = kv
        g1k, g1v = _take(k1, shf1), _take(v1, shf1)
        g2k, g2v = _take(k2, shf2), _take(v2, shf2)
        g3k, g3v = _take(k3, shf3), _take(v3, shf3)
        t01 = k0 >= g1k
        ak, av = jnp.where(t01, k0, g1k), jnp.where(t01, v0, g1v)
        t23 = g2k >= g3k
        bk, bv = jnp.where(t23, g2k, g3k), jnp.where(t23, g2v, g3v)
        ck, cv = jnp.where(lo8, ak, bk), jnp.where(lo8, av, bv)
        _, vf = plsc.sort_key_val(ck, cv, descending=True)
        rvec = jnp.full((LANES,), r, jnp.int32)
        w_v[r, :] = plsc.load_gather(p_v, [rvec, vf])
        i_v[r, :] = vf
    pltpu.sync_copy(w_v, wout_hbm.at[pl.ds(base, rows_per)])
    pltpu.sync_copy(i_v, iout_hbm.at[pl.ds(base, rows_per)])


def _sc_topk(probs, bias):
    n = probs.shape[0]
    rows_per = n // N_WORKERS
    mesh = plsc.VectorSubcoreMesh(core_axis_name="c", subcore_axis_name="s")
    return pl.kernel(
        functools.partial(_sc_topk_body, rows_per),
        out_type=[
            jax.ShapeDtypeStruct((n, LANES), jnp.float32),
            jax.ShapeDtypeStruct((n, LANES), jnp.int32),
        ],
        mesh=mesh,
        scratch_types=[
            pltpu.VMEM((rows_per, N_EXPERTS), jnp.float32),
            pltpu.VMEM((N_EXPERTS,), jnp.float32),
            pltpu.VMEM((rows_per, LANES), jnp.float32),
            pltpu.VMEM((rows_per, LANES), jnp.int32),
        ],
        compiler_params=pltpu.CompilerParams(needs_layout_passes=False),
    )(probs, bias)


@jax.jit
def kernel(x, weight, bias):
    probs = _tc_probs(x, weight)
    wpad, ipad = _sc_topk(probs, bias)
    return wpad[:, :TOP_K].astype(x.dtype), ipad[:, :TOP_K]


# speedup vs baseline: 1.0390x; 1.0025x over previous
import functools

import jax
import jax.numpy as jnp
from jax import lax
from jax.experimental import pallas as pl
from jax.experimental.pallas import tpu as pltpu
from jax.experimental.pallas import tpu_sc as plsc

DIM = 2048
N_EXPERTS = 64
TOP_K = 6
LANES = 16

BT = 1024
N_WORKERS = 32


def _mm_softmax_body(x_ref, w_ref, p_ref):
    sT = jax.lax.dot_general(
        w_ref[...], x_ref[...], (((1,), (1,)), ((), ())),
        preferred_element_type=jnp.float32,
    )
    m = jnp.max(sT, axis=0, keepdims=True)
    e = jnp.exp(sT - m)
    pT = e / jnp.sum(e, axis=0, keepdims=True)
    p_ref[...] = pT.T


def _tc_probs(x, weight):
    n = x.shape[0]
    return pl.pallas_call(
        _mm_softmax_body,
        grid=(n // BT,),
        in_specs=[
            pl.BlockSpec((BT, DIM), lambda i: (i, 0)),
            pl.BlockSpec((N_EXPERTS, DIM), lambda i: (0, 0)),
        ],
        out_specs=pl.BlockSpec((BT, N_EXPERTS), lambda i: (i, 0)),
        out_shape=jax.ShapeDtypeStruct((n, N_EXPERTS), jnp.float32),
    )(x, weight)


def _merge16(k1, v1, k2, v2):
    k2r = lax.rev(k2, (0,))
    v2r = lax.rev(v2, (0,))
    take1 = k1 >= k2r
    km = jnp.where(take1, k1, k2r)
    vm = jnp.where(take1, v1, v2r)
    return plsc.sort_key_val(km, vm, descending=True)


def _sc_topk_body(rows_per, p_hbm, bias_hbm, wout_hbm, iout_hbm,
                  p_v, bias_v, w_v, i_v):
    wid = lax.axis_index("s") * 2 + lax.axis_index("c")
    base = wid * rows_per
    pltpu.sync_copy(p_hbm.at[pl.ds(base, rows_per)], p_v)
    pltpu.sync_copy(bias_hbm, bias_v)
    biases = [bias_v[pl.ds(LANES * j, LANES)] for j in range(4)]
    iotas = [lax.iota(jnp.int32, LANES) + LANES * j for j in range(4)]

    iota = lax.iota(jnp.int32, LANES)
    lo8 = iota < 8
    shf1 = jnp.where(lo8, 7 - iota, 0)
    shf2 = jnp.where(lo8, 0, iota - 8)
    shf3 = 15 - iota

    def _take(x, idx):
        return x.at[idx].get(mode="promise_in_bounds")

    @plsc.parallel_loop(0, rows_per, unroll=1)
    def row_fn(r):
        segs = [p_v[r, pl.ds(LANES * j, LANES)] for j in range(4)]
        kv = [
            plsc.sort_key_val(segs[j] + biases[j], iotas[j], descending=True)
            for j in range(4)
        ]
        (k0, v0), (k1, v1), (k2, v2), (k3, v3) = kv
        g1k, g1v = _take(k1, shf1), _take(v1, shf1)
        g2k, g2v = _take(k2, shf2), _take(v2, shf2)
        g3k, g3v = _take(k3, shf3), _take(v3, shf3)
        t01 = k0 >= g1k
        ak, av = jnp.where(t01, k0, g1k), jnp.where(t01, v0, g1v)
        t23 = g2k >= g3k
        bk, bv = jnp.where(t23, g2k, g3k), jnp.where(t23, g2v, g3v)
        ck, cv = jnp.where(lo8, ak, bk), jnp.where(lo8, av, bv)
        _, vf = plsc.sort_key_val(ck, cv, descending=True)
        rvec = jnp.full((LANES,), r, jnp.int32)
        w_v[r, :] = plsc.load_gather(p_v, [rvec, vf])
        i_v[r, :] = vf
    pltpu.sync_copy(w_v, wout_hbm.at[pl.ds(base, rows_per)])
    pltpu.sync_copy(i_v, iout_hbm.at[pl.ds(base, rows_per)])


def _sc_topk(probs, bias):
    n = probs.shape[0]
    rows_per = n // N_WORKERS
    mesh = plsc.VectorSubcoreMesh(core_axis_name="c", subcore_axis_name="s")
    return pl.kernel(
        functools.partial(_sc_topk_body, rows_per),
        out_type=[
            jax.ShapeDtypeStruct((n, LANES), jnp.float32),
            jax.ShapeDtypeStruct((n, LANES), jnp.int32),
        ],
        mesh=mesh,
        scratch_types=[
            pltpu.VMEM((rows_per, N_EXPERTS), jnp.float32),
            pltpu.VMEM((N_EXPERTS,), jnp.float32),
            pltpu.VMEM((rows_per, LANES), jnp.float32),
            pltpu.VMEM((rows_per, LANES), jnp.int32),
        ],
        compiler_params=pltpu.CompilerParams(needs_layout_passes=False),
    )(probs, bias)


@jax.jit
def kernel(x, weight, bias):
    probs = _tc_probs(x, weight)
    wpad, ipad = _sc_topk(probs, bias)
    return wpad[:, :TOP_K].astype(x.dtype), ipad[:, :TOP_K]


# speedup vs baseline: 1.0427x; 1.0035x over previous
import functools

import jax
import jax.numpy as jnp
from jax import lax
from jax.experimental import pallas as pl
from jax.experimental.pallas import tpu as pltpu
from jax.experimental.pallas import tpu_sc as plsc

DIM = 2048
N_EXPERTS = 64
TOP_K = 6
LANES = 16

BT = 1024
N_WORKERS = 32


def _mm_softmax_body(x_ref, w_ref, p_ref):
    sT = jax.lax.dot_general(
        w_ref[...], x_ref[...], (((1,), (1,)), ((), ())),
        preferred_element_type=jnp.float32,
    )
    m = jnp.max(sT, axis=0, keepdims=True)
    e = jnp.exp(sT - m)
    pT = e / jnp.sum(e, axis=0, keepdims=True)
    p_ref[...] = pT.T


def _tc_probs(x, weight):
    n = x.shape[0]
    return pl.pallas_call(
        _mm_softmax_body,
        grid=(n // BT,),
        in_specs=[
            pl.BlockSpec((BT, DIM), lambda i: (i, 0)),
            pl.BlockSpec((N_EXPERTS, DIM), lambda i: (0, 0)),
        ],
        out_specs=pl.BlockSpec((BT, N_EXPERTS), lambda i: (i, 0)),
        out_shape=jax.ShapeDtypeStruct((n, N_EXPERTS), jnp.float32),
    )(x, weight)


def _merge16(k1, v1, k2, v2):
    k2r = lax.rev(k2, (0,))
    v2r = lax.rev(v2, (0,))
    take1 = k1 >= k2r
    km = jnp.where(take1, k1, k2r)
    vm = jnp.where(take1, v1, v2r)
    return plsc.sort_key_val(km, vm, descending=True)


def _sc_topk_body(rows_per, p_hbm, bias_hbm, wout_hbm, iout_hbm,
                  p_v, bias_v, w_v, i_v, sem0, sem1):
    wid = lax.axis_index("s") * 2 + lax.axis_index("c")
    base = wid * rows_per
    half = rows_per // 2
    cp0 = pltpu.async_copy(
        p_hbm.at[pl.ds(base, half)], p_v.at[pl.ds(0, half)], sem0)
    cp1 = pltpu.async_copy(
        p_hbm.at[pl.ds(base + half, half)], p_v.at[pl.ds(half, half)], sem1)
    pltpu.sync_copy(bias_hbm, bias_v)
    biases = [bias_v[pl.ds(LANES * j, LANES)] for j in range(4)]
    iotas = [lax.iota(jnp.int32, LANES) + LANES * j for j in range(4)]

    iota = lax.iota(jnp.int32, LANES)
    lo8 = iota < 8
    shf1 = jnp.where(lo8, 7 - iota, 0)
    shf2 = jnp.where(lo8, 0, iota - 8)
    shf3 = 15 - iota

    def _take(x, idx):
        return x.at[idx].get(mode="promise_in_bounds")

    def row_fn(r):
        segs = [p_v[r, pl.ds(LANES * j, LANES)] for j in range(4)]
        kv = [
            plsc.sort_key_val(segs[j] + biases[j], iotas[j], descending=True)
            for j in range(4)
        ]
        (k0, v0), (k1, v1), (k2, v2), (k3, v3) = kv
        g1k, g1v = _take(k1, shf1), _take(v1, shf1)
        g2k, g2v = _take(k2, shf2), _take(v2, shf2)
        g3k, g3v = _take(k3, shf3), _take(v3, shf3)
        t01 = k0 >= g1k
        ak, av = jnp.where(t01, k0, g1k), jnp.where(t01, v0, g1v)
        t23 = g2k >= g3k
        bk, bv = jnp.where(t23, g2k, g3k), jnp.where(t23, g2v, g3v)
        ck, cv = jnp.where(lo8, ak, bk), jnp.where(lo8, av, bv)
        _, vf = plsc.sort_key_val(ck, cv, descending=True)
        rvec = jnp.full((LANES,), r, jnp.int32)
        w_v[r, :] = plsc.load_gather(p_v, [rvec, vf])
        i_v[r, :] = vf

    cp0.wait()
    plsc.parallel_loop(0, half, unroll=2)(row_fn)
    cp1.wait()
    plsc.parallel_loop(half, rows_per, unroll=2)(row_fn)
    pltpu.sync_copy(w_v, wout_hbm.at[pl.ds(base, rows_per)])
    pltpu.sync_copy(i_v, iout_hbm.at[pl.ds(base, rows_per)])


def _sc_topk(probs, bias):
    n = probs.shape[0]
    rows_per = n // N_WORKERS
    mesh = plsc.VectorSubcoreMesh(core_axis_name="c", subcore_axis_name="s")
    return pl.kernel(
        functools.partial(_sc_topk_body, rows_per),
        out_type=[
            jax.ShapeDtypeStruct((n, LANES), jnp.float32),
            jax.ShapeDtypeStruct((n, LANES), jnp.int32),
        ],
        mesh=mesh,
        scratch_types=[
            pltpu.VMEM((rows_per, N_EXPERTS), jnp.float32),
            pltpu.VMEM((N_EXPERTS,), jnp.float32),
            pltpu.VMEM((rows_per, LANES), jnp.float32),
            pltpu.VMEM((rows_per, LANES), jnp.int32),
            pltpu.SemaphoreType.DMA,
            pltpu.SemaphoreType.DMA,
        ],
        compiler_params=pltpu.CompilerParams(needs_layout_passes=False),
    )(probs, bias)


@jax.jit
def kernel(x, weight, bias):
    probs = _tc_probs(x, weight)
    wpad, ipad = _sc_topk(probs, bias)
    return wpad[:, :TOP_K].astype(x.dtype), ipad[:, :TOP_K]
